# baseline (speedup 1.0000x reference)
import functools

import jax
import jax.numpy as jnp
from jax import lax
from jax.experimental import pallas as pl
from jax.experimental.pallas import tpu as pltpu
from jax.experimental.pallas import tpu_sc as plsc

N = 10000
D = 128
E = 320000
EPS = 1e-5

NC = 2
NS = 16
NW = NC * NS
CHUNK = 64
EPT = 10240
NCHUNK = EPT // CHUNK
EPAD = NW * EPT
NPAD = 10240
ROWS_PT = NPAD // NS

RBLK = 1024
NBLK = NPAD // RBLK


def _sc_body(x_hbm, ei_hbm, parts_hbm, degp_hbm,
             ei0, ei1, r0_v, r1_v, r2_v, r3_v, deg_v, acc_sh,
             sg0, sg1, sg2, sg3, ss0, ss1, ss2, ss3, sem_i0, sem_i1):
    c = lax.axis_index("c")
    s = lax.axis_index("s")
    wid = c * NS + s
    cbase = wid * NCHUNK

    zero16 = jnp.zeros((16,), jnp.float32)
    ones16 = jnp.ones((16,), jnp.float32)

    def _zdeg(i, carry):
        deg_v[pl.ds(i * 16, 16)] = zero16
        return carry
    lax.fori_loop(0, NPAD // 16, _zdeg, 0)

    def _zrows(i, carry):
        for j in range(D // 16):
            r0_v[i, pl.ds(j * 16, 16)] = zero16
        return carry
    lax.fori_loop(0, CHUNK, _zrows, 0)

    for k in range(ROWS_PT // CHUNK):
        pltpu.sync_copy(r0_v, acc_sh.at[pl.ds(s * ROWS_PT + k * CHUNK, CHUNK)])
    plsc.subcore_barrier()

    def _deg_update(ei, j):
        for k in range(CHUNK // 16):
            idx = ei[1, j, pl.ds(k * 16, 16)]
            plsc.addupdate_scatter(deg_v, [idx], ones16)

    rows = [r0_v, r1_v, r2_v, r3_v]
    sg = [sg0, sg1, sg2, sg3]
    ss = [ss0, ss1, ss2, ss3]

    pltpu.sync_copy(ei_hbm.at[:, pl.ds(cbase, 4)], ei0)
    pltpu.async_copy(ei_hbm.at[:, pl.ds(cbase + 4, 4)], ei1, sem_i1)
    pltpu.async_copy(x_hbm.at[ei0.at[0, 0]], rows[0], sg[0])
    pltpu.async_copy(x_hbm.at[ei0.at[0, 1]], rows[1], sg[1])

    def _gather(ei, j, b):
        pltpu.async_copy(x_hbm.at[ei.at[0, j]], rows[b], sg[b])

    def _wait_gather(ei, j, b):
        pltpu.make_async_copy(x_hbm.at[ei.at[0, j]], rows[b], sg[b]).wait()

    def _scatter(ei, j, b):
        pltpu.async_copy(rows[b], acc_sh.at[ei.at[1, j]], ss[b], add=True)

    def _wait_scatter(ei, j, b):
        pltpu.make_async_copy(rows[b], acc_sh.at[ei.at[1, j]], ss[b]).wait()

    def _half(u, ei_cur, ei_nxt, sem_nxt, refetch_nxt_at, nxt_base, first, last):
        _gather(ei_cur, 2, 2)
        _wait_gather(ei_cur, 0, 0)
        if first:
            @pl.when(u > 0)
            def _():
                _wait_scatter(ei_nxt, 3, 3)
                pltpu.async_copy(ei_hbm.at[:, pl.ds(refetch_nxt_at, 4)],
                                 ei_nxt, sem_nxt)
        else:
            _wait_scatter(ei_nxt, 3, 3)

            @pl.when(u < NCHUNK // 8 - 1)
            def _():
                pltpu.async_copy(ei_hbm.at[:, pl.ds(refetch_nxt_at, 4)],
                                 ei_nxt, sem_nxt)
        _scatter(ei_cur, 0, 0)
        _deg_update(ei_cur, 0)
        _gather(ei_cur, 3, 3)
        _wait_gather(ei_cur, 1, 1)
        _wait_scatter(ei_cur, 0, 0)
        _scatter(ei_cur, 1, 1)
        _deg_update(ei_cur, 1)
        if last:
            @pl.when(u < NCHUNK // 8 - 1)
            def _():
                pltpu.make_async_copy(ei_hbm.at[:, pl.ds(nxt_base, 4)],
                                      ei_nxt, sem_nxt).wait()
                _gather(ei_nxt, 0, 0)
        else:
            pltpu.make_async_copy(ei_hbm.at[:, pl.ds(nxt_base, 4)],
                                  ei_nxt, sem_nxt).wait()
            _gather(ei_nxt, 0, 0)
        _wait_gather(ei_cur, 2, 2)
        _wait_scatter(ei_cur, 1, 1)
        _scatter(ei_cur, 2, 2)
        _deg_update(ei_cur, 2)
        if last:
            @pl.when(u < NCHUNK // 8 - 1)
            def _():
                _gather(ei_nxt, 1, 1)
        else:
            _gather(ei_nxt, 1, 1)
        _wait_gather(ei_cur, 3, 3)
        _wait_scatter(ei_cur, 2, 2)
        _scatter(ei_cur, 3, 3)
        _deg_update(ei_cur, 3)

    def _superquad(u, carry):
        q0 = cbase + 8 * u
        _half(u, ei0, ei1, sem_i1, q0 + 4, q0 + 4, True, False)
        _half(u, ei1, ei0, sem_i0, q0 + 8, q0 + 8, False, True)
        return carry
    lax.fori_loop(0, NCHUNK // 8, _superquad, 0)
    pltpu.make_async_copy(rows[3], acc_sh.at[ei1.at[1, 3]], ss[3]).wait()

    plsc.subcore_barrier()

    for k in range(ROWS_PT // CHUNK):
        rr = s * ROWS_PT + k * CHUNK
        pltpu.sync_copy(acc_sh.at[pl.ds(rr, CHUNK)], r0_v)
        pltpu.sync_copy(r0_v, parts_hbm.at[c].at[pl.ds(rr, CHUNK)])
    pltpu.sync_copy(deg_v, degp_hbm.at[wid])


def _sc_gather_scatter(x, ei_p):
    mesh = plsc.VectorSubcoreMesh(core_axis_name="c", subcore_axis_name="s")
    return pl.kernel(
        _sc_body,
        mesh=mesh,
        out_type=[
            jax.ShapeDtypeStruct((NC, NPAD, D), jnp.float32),
            jax.ShapeDtypeStruct((NW, NPAD), jnp.float32),
        ],
        scratch_types=[
            pltpu.VMEM((2, 4, CHUNK), jnp.int32),
            pltpu.VMEM((2, 4, CHUNK), jnp.int32),
            pltpu.VMEM((CHUNK, D), jnp.float32),
            pltpu.VMEM((CHUNK, D), jnp.float32),
            pltpu.VMEM((CHUNK, D), jnp.float32),
            pltpu.VMEM((CHUNK, D), jnp.float32),
            pltpu.VMEM((NPAD,), jnp.float32),
            pltpu.VMEM_SHARED((NPAD, D), jnp.float32),
            pltpu.SemaphoreType.DMA,
            pltpu.SemaphoreType.DMA,
            pltpu.SemaphoreType.DMA,
            pltpu.SemaphoreType.DMA,
            pltpu.SemaphoreType.DMA,
            pltpu.SemaphoreType.DMA,
            pltpu.SemaphoreType.DMA,
            pltpu.SemaphoreType.DMA,
            pltpu.SemaphoreType.DMA,
            pltpu.SemaphoreType.DMA,
        ],
        compiler_params=pltpu.CompilerParams(needs_layout_passes=False),
    )(x, ei_p)


def _compute_h(parts_ref, degp_ref, x_ref, w_ref, p_ref):
    deg = jnp.sum(degp_ref[...], axis=0)
    summed = parts_ref[0] + parts_ref[1]
    agg = summed / jnp.maximum(deg, 1.0)[:, None]
    xb = x_ref[...]
    hm = (jnp.dot(agg, w_ref[0], preferred_element_type=jnp.float32)
          + jnp.dot(xb, w_ref[1], preferred_element_type=jnp.float32)
          + p_ref[0, :][None, :])
    hs = jnp.dot(xb, w_ref[2], preferred_element_type=jnp.float32) + p_ref[1, :][None, :]
    return hm, hs


def _tc_stats_body(parts_ref, degp_ref, x_ref, w_ref, p_ref, stats_ref):
    i = pl.program_id(0)

    @pl.when(i == 0)
    def _():
        stats_ref[...] = jnp.zeros_like(stats_ref)
    hm, hs = _compute_h(parts_ref, degp_ref, x_ref, w_ref, p_ref)
    row = i * RBLK + lax.broadcasted_iota(jnp.int32, (RBLK, 1), 0)
    valid = row < N
    hm = jnp.where(valid, hm, 0.0)
    hs = jnp.where(valid, hs, 0.0)
    stats_ref[0, :] += jnp.sum(hm, axis=0)
    stats_ref[1, :] += jnp.sum(hm * hm, axis=0)
    stats_ref[2, :] += jnp.sum(hs, axis=0)
    stats_ref[3, :] += jnp.sum(hs * hs, axis=0)


def _tc_final_body(parts_ref, degp_ref, x_ref, w_ref, p_ref, stats_ref, out_ref):
    hm, hs = _compute_h(parts_ref, degp_ref, x_ref, w_ref, p_ref)
    ninv = 1.0 / N
    a_m = p_ref[4, :]
    a_s = p_ref[7, :]
    mu_m = stats_ref[0, :] * ninv
    var_m = stats_ref[1, :] * ninv - mu_m * mu_m * (2.0 * a_m - a_m * a_m)
    mu_s = stats_ref[2, :] * ninv
    var_s = stats_ref[3, :] * ninv - mu_s * mu_s * (2.0 * a_s - a_s * a_s)
    scale_m = p_ref[2, :] * lax.rsqrt(var_m + EPS)
    scale_s = p_ref[5, :] * lax.rsqrt(var_s + EPS)
    ym = (hm - (a_m * mu_m)[None, :]) * scale_m[None, :] + p_ref[3, :][None, :]
    ys = (hs - (a_s * mu_s)[None, :]) * scale_s[None, :] + p_ref[6, :][None, :]
    z = ym + ys
    out_ref[...] = jnp.where(z > 0, z, jnp.exp(z) - 1.0)


_TC_IN_SPECS = [
    pl.BlockSpec((NC, RBLK, D), lambda i: (0, i, 0)),
    pl.BlockSpec((NW, RBLK), lambda i: (0, i)),
    pl.BlockSpec((RBLK, D), lambda i: (i, 0)),
    pl.BlockSpec((3, D, D), lambda i: (0, 0, 0)),
    pl.BlockSpec((8, D), lambda i: (0, 0)),
]


def _tc_fuse(parts, degp, x, wstack, pvec, interpret=False):
    stats = pl.pallas_call(
        _tc_stats_body,
        grid=(NBLK,),
        in_specs=_TC_IN_SPECS,
        out_specs=pl.BlockSpec((8, D), lambda i: (0, 0)),
        out_shape=jax.ShapeDtypeStruct((8, D), jnp.float32),
        interpret=interpret,
    )(parts, degp, x, wstack, pvec)
    return pl.pallas_call(
        _tc_final_body,
        grid=(NBLK,),
        in_specs=_TC_IN_SPECS + [pl.BlockSpec((8, D), lambda i: (0, 0))],
        out_specs=pl.BlockSpec((RBLK, D), lambda i: (i, 0)),
        out_shape=jax.ShapeDtypeStruct((NPAD, D), jnp.float32),
        interpret=interpret,
    )(parts, degp, x, wstack, pvec, stats)


def kernel(x, edge_index, batch, Wl, bl, Wr, Wskip, bskip,
           bn_weight, bn_bias, bn_alpha, sk_weight, sk_bias, sk_alpha):
    pad = EPAD - E
    pad_blk = jnp.stack([jnp.zeros((pad,), jnp.int32),
                         jnp.full((pad,), N, jnp.int32)])
    ei_p = jnp.concatenate([edge_index, pad_blk], axis=1)
    ei_p = ei_p.reshape(2, EPAD // CHUNK, CHUNK)

    parts, degp = _sc_gather_scatter(x, ei_p)

    x_p = jnp.concatenate([x, jnp.zeros((NPAD - N, D), jnp.float32)])
    wstack = jnp.stack([Wl.T, Wr.T, Wskip.T])
    pvec = jnp.stack([bl, bskip, bn_weight, bn_bias, bn_alpha,
                      sk_weight, sk_bias, sk_alpha])
    return _tc_fuse(parts, degp, x_p, wstack, pvec)[:N]
